# baseline (speedup 1.0000x reference)
import jax
import jax.numpy as jnp
from jax.experimental import pallas as pl
from jax.experimental.pallas import tpu as pltpu

_B, _S, _H = 16, 2048, 1024
_L, _M = 32, 4


def _body(spans_ref, seq_ref, w_ref, bias_ref,
          logits_ref, hid_ref, best_ref):
    bb = pl.program_id(0)

    sp = spans_ref[0]
    s = sp[:, 0:1]
    e = sp[:, 1:2]

    pos = jax.lax.broadcasted_iota(jnp.int32, (_L * _M, _S), 1)
    maskf = ((pos >= s) & (pos <= e)).astype(jnp.bfloat16)

    x = seq_ref[0].astype(jnp.bfloat16)
    sums = jax.lax.dot_general(
        maskf, x,
        dimension_numbers=(((1,), (0,)), ((), ())),
        preferred_element_type=jnp.float32,
    )

    counts = (e - s + 1).astype(jnp.float32)
    means = sums / counts
    mr = means.reshape(_L, _M, _H)
    hid = jnp.maximum(
        jnp.maximum(mr[:, 0], mr[:, 1]),
        jnp.maximum(mr[:, 2], mr[:, 3]),
    )
    hid_ref[0] = hid

    logits = jax.lax.dot_general(
        w_ref[...].astype(jnp.bfloat16), hid.astype(jnp.bfloat16),
        dimension_numbers=(((1,), (1,)), ((), ())),
        preferred_element_type=jnp.float32,
    ) + bias_ref[0]

    maxv = jnp.max(logits, axis=1, keepdims=True)
    ii = jax.lax.broadcasted_iota(jnp.int32, (1, _L), 1)
    best = jnp.min(jnp.where(logits == maxv, ii, jnp.int32(2**30)),
                   axis=1, keepdims=True)
    logits_ref[pl.ds(bb, 1), :] = logits
    best_ref[bb] = best[0, 0]


def kernel(seq_hiddens, links_spans, W, b):
    B, S, H = seq_hiddens.shape
    L, M = links_spans.shape[1], links_spans.shape[2]
    sp = links_spans.reshape(B, L * M, 2).astype(jnp.int32)

    out = pl.pallas_call(
        _body,
        grid=(B,),
        in_specs=[
            pl.BlockSpec((1, M * L, 2), lambda bb: (bb, 0, 0)),
            pl.BlockSpec((1, S, H), lambda bb: (bb, 0, 0)),
            pl.BlockSpec((1, H), lambda bb: (0, 0)),
            pl.BlockSpec(memory_space=pltpu.SMEM),
        ],
        out_specs=[
            pl.BlockSpec((B, L), lambda bb: (0, 0)),
            pl.BlockSpec((1, L, H), lambda bb: (bb, 0, 0)),
            pl.BlockSpec(memory_space=pltpu.SMEM),
        ],
        out_shape=[
            jax.ShapeDtypeStruct((B, L), jnp.float32),
            jax.ShapeDtypeStruct((B, L, H), jnp.float32),
            jax.ShapeDtypeStruct((B,), jnp.int32),
        ],
        compiler_params=pltpu.CompilerParams(
            dimension_semantics=("arbitrary",),
        ),
    )(sp, seq_hiddens, W, b)

    return (out[0], out[1], out[2])


# speedup vs baseline: 1.0548x; 1.0548x over previous
import jax
import jax.numpy as jnp
from jax.experimental import pallas as pl
from jax.experimental.pallas import tpu as pltpu

_B, _S, _H = 16, 2048, 1024
_L, _M = 32, 4


def _body(spans_ref, seq_ref, w_ref, bias_ref,
          logits_ref, hid_ref, best_ref):
    bb = pl.program_id(0)

    sp = spans_ref[0]
    s = sp[:, 0:1]
    e = sp[:, 1:2]

    pos = jax.lax.broadcasted_iota(jnp.int32, (_L * _M, _S), 1)
    maskf = ((pos >= s) & (pos <= e)).astype(jnp.bfloat16)

    x = seq_ref[0].astype(jnp.bfloat16)
    sums = jax.lax.dot_general(
        maskf, x,
        dimension_numbers=(((1,), (0,)), ((), ())),
        preferred_element_type=jnp.float32,
    )

    counts = (e - s + 1).astype(jnp.float32)
    means = sums / counts
    hid = jnp.maximum(
        jnp.maximum(means[0 * _L:1 * _L], means[1 * _L:2 * _L]),
        jnp.maximum(means[2 * _L:3 * _L], means[3 * _L:4 * _L]),
    )
    hid_ref[0] = hid

    logits = jax.lax.dot_general(
        w_ref[...].astype(jnp.bfloat16), hid.astype(jnp.bfloat16),
        dimension_numbers=(((1,), (1,)), ((), ())),
        preferred_element_type=jnp.float32,
    ) + bias_ref[0]

    maxv = jnp.max(logits, axis=1, keepdims=True)
    ii = jax.lax.broadcasted_iota(jnp.int32, (1, _L), 1)
    best = jnp.min(jnp.where(logits == maxv, ii, jnp.int32(2**30)),
                   axis=1, keepdims=True)
    logits_ref[pl.ds(bb, 1), :] = logits
    best_ref[bb] = best[0, 0]


def kernel(seq_hiddens, links_spans, W, b):
    B, S, H = seq_hiddens.shape
    L, M = links_spans.shape[1], links_spans.shape[2]
    sp = jnp.transpose(links_spans, (0, 2, 1, 3)).reshape(B, M * L, 2)
    sp = sp.astype(jnp.int32)

    out = pl.pallas_call(
        _body,
        grid=(B,),
        in_specs=[
            pl.BlockSpec((1, M * L, 2), lambda bb: (bb, 0, 0)),
            pl.BlockSpec((1, S, H), lambda bb: (bb, 0, 0)),
            pl.BlockSpec((1, H), lambda bb: (0, 0)),
            pl.BlockSpec(memory_space=pltpu.SMEM),
        ],
        out_specs=[
            pl.BlockSpec((B, L), lambda bb: (0, 0)),
            pl.BlockSpec((1, L, H), lambda bb: (bb, 0, 0)),
            pl.BlockSpec(memory_space=pltpu.SMEM),
        ],
        out_shape=[
            jax.ShapeDtypeStruct((B, L), jnp.float32),
            jax.ShapeDtypeStruct((B, L, H), jnp.float32),
            jax.ShapeDtypeStruct((B,), jnp.int32),
        ],
        compiler_params=pltpu.CompilerParams(
            dimension_semantics=("arbitrary",),
        ),
    )(sp, seq_hiddens, W, b)

    return (out[0], out[1], out[2])
